# baseline (speedup 1.0000x reference)
import functools

import jax
import jax.numpy as jnp
from jax import lax
from jax.experimental import pallas as pl
from jax.experimental.pallas import tpu as pltpu
from jax.experimental.pallas import tpu_sc as plsc

_NC = 2
_NS = 16
_L = 16
_NW = _NC * _NS

_ROWS = 128
_COLS = 32768
_RPW = _ROWS // _NW
_CW = 4096
_NCHUNK = _COLS // _CW
_UNROLL = 4
_NEG = float("-inf")


def _insert5(stack, v):
    out = []
    for t in range(5):
        hi = jnp.maximum(stack[t], v)
        v = jnp.minimum(stack[t], v)
        out.append(hi)
    return out


_GV = 16
_GROUPS = _CW // (_GV * _L)


def _sc_body(x_hbm, out_hbm, buf0, buf1, obuf, sem0, sem1):
    wid = lax.axis_index("s") * _NC + lax.axis_index("c")
    row0 = wid * _RPW

    bufs = (buf0, buf1)
    sems = (sem0, sem1)

    def dma(k):
        return pltpu.make_async_copy(
            x_hbm.at[pl.ds(row0, _RPW), pl.ds(k * _CW, _CW)],
            bufs[k % 2], sems[k % 2])

    neg = jnp.full((_L,), _NEG, dtype=jnp.float32)
    iota = lax.iota(jnp.int32, _L)
    carry = (neg, neg, neg, neg, neg, neg)

    dma(0).start()
    for k in range(_NCHUNK):
        if k + 1 < _NCHUNK:
            dma(k + 1).start()
        dma(k).wait()
        buf = bufs[k % 2]

        def step(i, c, buf=buf):
            for j in range(_RPW):
                base = i * _GV * _L
                vs = [buf[j, pl.ds(base + t * _L, _L)] for t in range(_GV)]
                while len(vs) > 1:
                    vs = [jnp.maximum(vs[p], vs[p + 1])
                          for p in range(0, len(vs) - 1, 2)] + (
                              [vs[-1]] if len(vs) % 2 else [])
                hit = jnp.any(vs[0] > c[5])

                def rescan(op, j=j, base=base, buf=buf):
                    def ins(t, s, j=j, base=base, buf=buf):
                        v = buf[j, pl.ds(base + t * _L, _L)]
                        return tuple(_insert5(list(s), v))
                    st = lax.fori_loop(0, _GV, ins, op[:5])
                    srt = jnp.sort(st[0])
                    thr = jnp.max(jnp.where(iota == _L - 5, srt, _NEG))
                    return st + (jnp.zeros((_L,), jnp.float32) + thr,)

                c = lax.cond(hit, rescan, lambda op: op, c)
            return c

        carry = lax.fori_loop(0, _GROUPS, step, carry)

    for t in range(5):
        obuf[pl.ds(t * _L, _L)] = carry[t]
    pltpu.sync_copy(obuf, out_hbm.at[wid])


@jax.jit
def _sc_topk_candidates(x):
    mesh = plsc.VectorSubcoreMesh(core_axis_name="c", subcore_axis_name="s",
                                  num_cores=_NC, num_subcores=_NS)
    k = pl.kernel(
        _sc_body,
        out_type=jax.ShapeDtypeStruct((_NW, 5 * _L), jnp.float32),
        mesh=mesh,
        scratch_types=[
            pltpu.VMEM((_RPW, _CW), jnp.float32),
            pltpu.VMEM((_RPW, _CW), jnp.float32),
            pltpu.VMEM((5 * _L,), jnp.float32),
            pltpu.SemaphoreType.DMA,
            pltpu.SemaphoreType.DMA,
        ],
        compiler_params=pltpu.CompilerParams(needs_layout_passes=False),
    )
    return k(x)


def _merge_body(c_ref, yt_ref, o_ref):
    x = c_ref[...]
    r, l = x.shape
    li = (lax.broadcasted_iota(jnp.int32, (r, l), 0) * l
          + lax.broadcasted_iota(jnp.int32, (r, l), 1))
    prod = jnp.float32(1.0)
    for _ in range(5):
        t = jnp.max(x)
        sel = jnp.where(x == t, li, jnp.int32(2 ** 30))
        fi = jnp.min(sel)
        x = jnp.where(li == fi, _NEG, x)
        prod = prod * (jnp.float32(1.0) - t)
    y_site = jnp.float32(1.0) - prod
    d = y_site - yt_ref[0, 0]
    o_ref[0, 0] = d * d


@jax.jit
def _merge_loss(cands, y_true):
    return pl.pallas_call(
        _merge_body,
        out_shape=jax.ShapeDtypeStruct((1, 1), jnp.float32),
        in_specs=[
            pl.BlockSpec(memory_space=pltpu.VMEM),
            pl.BlockSpec(memory_space=pltpu.SMEM),
        ],
        out_specs=pl.BlockSpec(memory_space=pltpu.SMEM),
    )(cands, y_true)


def kernel(y_pred, y_true):
    cands = _sc_topk_candidates(y_pred)
    loss = _merge_loss(cands, y_true.reshape(1, 1))
    return loss.reshape(1)


# speedup vs baseline: 1.1612x; 1.1612x over previous
import functools

import jax
import jax.numpy as jnp
from jax import lax
from jax.experimental import pallas as pl
from jax.experimental.pallas import tpu as pltpu
from jax.experimental.pallas import tpu_sc as plsc

_NC = 2
_NS = 16
_L = 16
_NW = _NC * _NS

_ROWS = 128
_COLS = 32768
_RPW = _ROWS // _NW
_CW = 4096
_NCHUNK = _COLS // _CW
_UNROLL = 4
_NEG = float("-inf")


def _insert5(stack, v):
    out = []
    for t in range(5):
        hi = jnp.maximum(stack[t], v)
        v = jnp.minimum(stack[t], v)
        out.append(hi)
    return out


_GV = 16
_GROUPS = _CW // (_GV * _L)
_NGRP = _NCHUNK * _RPW * _GROUPS


def _sc_body(x_hbm, out_hbm, buf0, buf1, sums, cand_ids, candbuf, obuf,
             sem0, sem1, csem):
    wid = lax.axis_index("s") * _NC + lax.axis_index("c")
    row0 = wid * _RPW

    bufs = (buf0, buf1)
    sems = (sem0, sem1)

    def dma(k):
        return pltpu.make_async_copy(
            x_hbm.at[pl.ds(row0, _RPW), pl.ds(k * _CW, _CW)],
            bufs[k % 2], sems[k % 2])

    neg = jnp.full((_L,), _NEG, dtype=jnp.float32)

    S = (neg, neg, neg, neg, neg)
    dma(0).start()
    for k in range(_NCHUNK):
        if k + 1 < _NCHUNK:
            dma(k + 1).start()
        dma(k).wait()
        buf = bufs[k % 2]
        for j in range(_RPW):
            soff = (k * _RPW + j) * _GROUPS * _L

            def step(i, c, buf=buf, j=j, soff=soff):
                base = i * _GV * _L
                vs = [buf[j, pl.ds(base + t * _L, _L)] for t in range(_GV)]
                while len(vs) > 1:
                    vs = [jnp.maximum(vs[p], vs[p + 1])
                          for p in range(0, len(vs) - 1, 2)] + (
                              [vs[-1]] if len(vs) % 2 else [])
                sums[pl.ds(soff + i * _L, _L)] = vs[0]
                return tuple(_insert5(list(c), vs[0]))

            S = lax.fori_loop(0, _GROUPS, step, S)

    a0, a1, a2, a3, a4 = S
    thr = None
    for _ in range(5):
        thr = jnp.max(a0)
        eq = a0 == thr
        cs = plsc.cumsum(eq.astype(jnp.int32))
        first = jnp.logical_and(eq, cs == 1)
        a0 = jnp.where(first, a1, a0)
        a1 = jnp.where(first, a2, a1)
        a2 = jnp.where(first, a3, a2)
        a3 = jnp.where(first, a4, a3)
        a4 = jnp.where(first, neg, a4)

    def collect(g, p):
        m = sums[pl.ds(g * _L, _L)]
        hit = jnp.any(m >= thr)
        cand_ids[p] = g
        return p + hit.astype(jnp.int32)

    n = lax.fori_loop(0, _NGRP, collect, jnp.int32(0))

    F = (neg, neg, neg, neg, neg)

    def rescan(c, f):
        g = cand_ids[c]
        j = (g >> 4) & (_RPW - 1)
        col = (g >> 6) * _CW + (g & (_GROUPS - 1)) * _GV * _L
        cp = pltpu.make_async_copy(
            x_hbm.at[pl.ds(row0 + j, 1), pl.ds(col, _GV * _L)],
            candbuf, csem)
        cp.start()
        cp.wait()
        f = list(f)
        for t in range(_GV):
            f = _insert5(f, candbuf[0, pl.ds(t * _L, _L)])
        return tuple(f)

    F = lax.fori_loop(0, n, rescan, F)

    for t in range(5):
        obuf[pl.ds(t * _L, _L)] = F[t]
    pltpu.sync_copy(obuf, out_hbm.at[wid])


@jax.jit
def _sc_topk_candidates(x):
    mesh = plsc.VectorSubcoreMesh(core_axis_name="c", subcore_axis_name="s",
                                  num_cores=_NC, num_subcores=_NS)
    k = pl.kernel(
        _sc_body,
        out_type=jax.ShapeDtypeStruct((_NW, 5 * _L), jnp.float32),
        mesh=mesh,
        scratch_types=[
            pltpu.VMEM((_RPW, _CW), jnp.float32),
            pltpu.VMEM((_RPW, _CW), jnp.float32),
            pltpu.VMEM((_NGRP * _L,), jnp.float32),
            pltpu.SMEM((_NGRP,), jnp.int32),
            pltpu.VMEM((1, _GV * _L), jnp.float32),
            pltpu.VMEM((5 * _L,), jnp.float32),
            pltpu.SemaphoreType.DMA,
            pltpu.SemaphoreType.DMA,
            pltpu.SemaphoreType.DMA,
        ],
        compiler_params=pltpu.CompilerParams(needs_layout_passes=False),
    )
    return k(x)


def _merge_body(c_ref, yt_ref, o_ref):
    x = c_ref[...]
    r, l = x.shape
    li = (lax.broadcasted_iota(jnp.int32, (r, l), 0) * l
          + lax.broadcasted_iota(jnp.int32, (r, l), 1))
    prod = jnp.float32(1.0)
    for _ in range(5):
        t = jnp.max(x)
        sel = jnp.where(x == t, li, jnp.int32(2 ** 30))
        fi = jnp.min(sel)
        x = jnp.where(li == fi, _NEG, x)
        prod = prod * (jnp.float32(1.0) - t)
    y_site = jnp.float32(1.0) - prod
    d = y_site - yt_ref[0, 0]
    o_ref[0, 0] = d * d


@jax.jit
def _merge_loss(cands, y_true):
    return pl.pallas_call(
        _merge_body,
        out_shape=jax.ShapeDtypeStruct((1, 1), jnp.float32),
        in_specs=[
            pl.BlockSpec(memory_space=pltpu.VMEM),
            pl.BlockSpec(memory_space=pltpu.SMEM),
        ],
        out_specs=pl.BlockSpec(memory_space=pltpu.SMEM),
    )(cands, y_true)


def kernel(y_pred, y_true):
    cands = _sc_topk_candidates(y_pred)
    loss = _merge_loss(cands, y_true.reshape(1, 1))
    return loss.reshape(1)
